# baseline (speedup 1.0000x reference)
import functools

import jax
import jax.numpy as jnp
import numpy as np
from jax import lax
from jax.experimental import pallas as pl
from jax.experimental.pallas import tpu as pltpu
from jax.experimental.pallas import tpu_sc as plsc

_VOCAB = 1_000_000
_VOCAB_PAD = 1_048_576
_TROWS = 8192
_TGRID = 8

_B, _N = 16384, 200
_TOTAL = _B * _N
_NC, _NS = 2, 16
_NW = _NC * _NS
_LANES = 128
_PER_W = _TOTAL // _NW
_CROWS = 64
_CHUNK = _CROWS * _N
_ROWS_PER_W = _B // _NW
_NCHUNKS = _ROWS_PER_W // _CROWS


_E5 = float(np.exp(5.0))


def _table_body(total_ref, counts_ref, out_ref):
    total = total_ref[0, 0]
    freq = counts_ref[...] / total
    out_ref[...] = 1.0 / (1.0 + (freq + 1e-9) * _E5)


def _transform_table(counts_padded, total_tokens):
    blk = _TROWS // _TGRID
    return pl.pallas_call(
        _table_body,
        grid=(_TGRID,),
        in_specs=[
            pl.BlockSpec(memory_space=pltpu.SMEM),
            pl.BlockSpec((blk, _LANES), lambda i: (i, 0)),
        ],
        out_specs=pl.BlockSpec((blk, _LANES), lambda i: (i, 0)),
        out_shape=jax.ShapeDtypeStruct((_TROWS, _LANES), jnp.float32),
    )(jnp.reshape(total_tokens, (1, 1)), counts_padded)


def _gather_body(x_hbm, table_hbm, out_hbm, tab_s,
                 idx0, idx1, val0, val1,
                 tsem, isem0, isem1, gsem, osem0, osem1):
    sid = lax.axis_index("s")
    wid = sid * _NC + lax.axis_index("c")
    base = wid * _PER_W

    idx_bufs, val_bufs = [idx0, idx1], [val0, val1]
    isems, osems = [isem0, isem1], [osem0, osem1]

    @pl.when(sid == 0)
    def _():
        pltpu.async_copy(table_hbm, tab_s, tsem).wait()

    idx_loads = [
        pltpu.make_async_copy(
            x_hbm.at[pl.ds(base + c * _CHUNK, _CHUNK)], idx_bufs[c % 2],
            isems[c % 2])
        for c in range(_NCHUNKS)
    ]
    out_stores = [
        pltpu.make_async_copy(
            val_bufs[c % 2], out_hbm.at[pl.ds(base + c * _CHUNK, _CHUNK)],
            osems[c % 2])
        for c in range(_NCHUNKS)
    ]

    idx_loads[0].start()
    plsc.subcore_barrier()

    for c in range(_NCHUNKS):
        b = c % 2
        if c >= 2:
            out_stores[c - 2].wait()
        idx_loads[c].wait()
        if c + 1 < _NCHUNKS:
            idx_loads[c + 1].start()
        def _fire(k, carry):
            pltpu.make_async_copy(
                tab_s.at[idx_bufs[b].at[pl.ds(k * 128, 128)]],
                val_bufs[b].at[pl.ds(k * 128, 128)], gsem).start()
            return carry

        lax.fori_loop(0, _CHUNK // 128, _fire, 0, unroll=8)

        def _drain(k, carry):
            pltpu.make_async_copy(
                tab_s.at[idx_bufs[b].at[pl.ds(0, 128)]],
                val_bufs[b].at[pl.ds(0, 128)], gsem).wait()
            return carry

        lax.fori_loop(0, _CHUNK // 128, _drain, 0, unroll=8)
        out_stores[c].start()

    out_stores[_NCHUNKS - 2].wait()
    out_stores[_NCHUNKS - 1].wait()


_gather = pl.kernel(
    _gather_body,
    out_type=jax.ShapeDtypeStruct((_TOTAL,), jnp.float32),
    mesh=plsc.VectorSubcoreMesh(core_axis_name="c", subcore_axis_name="s"),
    scratch_types=[
        pltpu.VMEM_SHARED((_VOCAB_PAD,), jnp.float32),
        pltpu.VMEM((_CHUNK,), jnp.int32),
        pltpu.VMEM((_CHUNK,), jnp.int32),
        pltpu.VMEM((_CHUNK,), jnp.float32),
        pltpu.VMEM((_CHUNK,), jnp.float32),
        pltpu.SemaphoreType.DMA,
        pltpu.SemaphoreType.DMA,
        pltpu.SemaphoreType.DMA,
        pltpu.SemaphoreType.DMA,
        pltpu.SemaphoreType.DMA,
        pltpu.SemaphoreType.DMA,
    ],
)


@jax.jit
def kernel(x, token_counts, total_tokens):
    counts_padded = jnp.concatenate(
        [token_counts, jnp.ones((_VOCAB_PAD - _VOCAB,), jnp.float32)]
    ).reshape(_TROWS, _LANES)
    table = _transform_table(counts_padded, total_tokens).reshape(-1)
    out = _gather(x.reshape(-1), table)
    return out.reshape(_B, _N)


# speedup vs baseline: 1.3979x; 1.3979x over previous
import functools

import jax
import jax.numpy as jnp
import numpy as np
from jax import lax
from jax.experimental import pallas as pl
from jax.experimental.pallas import tpu as pltpu
from jax.experimental.pallas import tpu_sc as plsc

_VOCAB = 1_000_000
_VOCAB_PAD = 1_048_576
_TROWS = 8192
_TGRID = 8
_LANES = 128

_B, _N = 16384, 200
_NA, _NB = 128, 72
_TOTAL = _B * _N
_NC, _NS = 2, 16
_NW = _NC * _NS
_CROWS = 64
_ROWS_PER_W = _B // _NW
_NCHUNKS = _ROWS_PER_W // _CROWS

_E5 = float(np.exp(5.0))


def _table_body(total_ref, counts_ref, out_ref):
    total = total_ref[0, 0]
    freq = counts_ref[...] / total
    out_ref[...] = 1.0 / (1.0 + (freq + 1e-9) * _E5)


def _transform_table(counts_padded, total_tokens):
    blk = _TROWS // _TGRID
    return pl.pallas_call(
        _table_body,
        grid=(_TGRID,),
        in_specs=[
            pl.BlockSpec(memory_space=pltpu.SMEM),
            pl.BlockSpec((blk, _LANES), lambda i: (i, 0)),
        ],
        out_specs=pl.BlockSpec((blk, _LANES), lambda i: (i, 0)),
        out_shape=jax.ShapeDtypeStruct((_TROWS, _LANES), jnp.float32),
    )(jnp.reshape(total_tokens, (1, 1)), counts_padded)


def _gather_body(x_hbm, table_hbm, out_hbm, tab_s,
                 idxa0, idxa1, idxb0, idxb1,
                 vala0, vala1, valb0, valb1,
                 tsem, isem0, isem1, gsem, osem0, osem1):
    sid = lax.axis_index("s")
    wid = sid * _NC + lax.axis_index("c")
    rbase = wid * _ROWS_PER_W

    idxa, idxb = [idxa0, idxa1], [idxb0, idxb1]
    vala, valb = [vala0, vala1], [valb0, valb1]
    isems, osems = [isem0, isem1], [osem0, osem1]

    @pl.when(sid == 0)
    def _():
        pltpu.async_copy(table_hbm, tab_s, tsem).wait()

    def rows(c):
        return pl.ds(rbase + c * _CROWS, _CROWS)

    idx_loads = [
        (pltpu.make_async_copy(x_hbm.at[rows(c), pl.ds(0, _NA)],
                               idxa[c % 2], isems[c % 2]),
         pltpu.make_async_copy(x_hbm.at[rows(c), pl.ds(_NA, _NB)],
                               idxb[c % 2], isems[c % 2]))
        for c in range(_NCHUNKS)
    ]
    out_stores = [
        (pltpu.make_async_copy(vala[c % 2], out_hbm.at[rows(c), pl.ds(0, _NA)],
                               osems[c % 2]),
         pltpu.make_async_copy(valb[c % 2], out_hbm.at[rows(c), pl.ds(_NA, _NB)],
                               osems[c % 2]))
        for c in range(_NCHUNKS)
    ]

    idx_loads[0][0].start()
    idx_loads[0][1].start()
    plsc.subcore_barrier()

    for c in range(_NCHUNKS):
        b = c % 2
        if c >= 2:
            out_stores[c - 2][0].wait()
            out_stores[c - 2][1].wait()
        idx_loads[c][0].wait()
        idx_loads[c][1].wait()
        if c + 1 < _NCHUNKS:
            idx_loads[c + 1][0].start()
            idx_loads[c + 1][1].start()

        def _fire(r, carry):
            pltpu.make_async_copy(
                tab_s.at[idxa[b].at[r]], vala[b].at[r], gsem).start()
            pltpu.make_async_copy(
                tab_s.at[idxb[b].at[r]], valb[b].at[r], gsem).start()
            return carry

        lax.fori_loop(0, _CROWS, _fire, 0, unroll=8)

        def _drain(r, carry):
            pltpu.make_async_copy(
                tab_s.at[idxa[b].at[0]], vala[b].at[0], gsem).wait()
            pltpu.make_async_copy(
                tab_s.at[idxb[b].at[0]], valb[b].at[0], gsem).wait()
            return carry

        lax.fori_loop(0, _CROWS, _drain, 0, unroll=8)
        out_stores[c][0].start()
        out_stores[c][1].start()

    for c in (_NCHUNKS - 2, _NCHUNKS - 1):
        out_stores[c][0].wait()
        out_stores[c][1].wait()


_gather = pl.kernel(
    _gather_body,
    out_type=jax.ShapeDtypeStruct((_B, _N), jnp.float32),
    mesh=plsc.VectorSubcoreMesh(core_axis_name="c", subcore_axis_name="s"),
    scratch_types=[
        pltpu.VMEM_SHARED((_VOCAB_PAD,), jnp.float32),
        pltpu.VMEM((_CROWS, _NA), jnp.int32),
        pltpu.VMEM((_CROWS, _NA), jnp.int32),
        pltpu.VMEM((_CROWS, _NB), jnp.int32),
        pltpu.VMEM((_CROWS, _NB), jnp.int32),
        pltpu.VMEM((_CROWS, _NA), jnp.float32),
        pltpu.VMEM((_CROWS, _NA), jnp.float32),
        pltpu.VMEM((_CROWS, _NB), jnp.float32),
        pltpu.VMEM((_CROWS, _NB), jnp.float32),
        pltpu.SemaphoreType.DMA,
        pltpu.SemaphoreType.DMA,
        pltpu.SemaphoreType.DMA,
        pltpu.SemaphoreType.DMA,
        pltpu.SemaphoreType.DMA,
        pltpu.SemaphoreType.DMA,
    ],
)


@jax.jit
def kernel(x, token_counts, total_tokens):
    counts_padded = jnp.concatenate(
        [token_counts, jnp.ones((_VOCAB_PAD - _VOCAB,), jnp.float32)]
    ).reshape(_TROWS, _LANES)
    table = _transform_table(counts_padded, total_tokens).reshape(-1)
    return _gather(x, table)


# speedup vs baseline: 2.0227x; 1.4470x over previous
import functools

import jax
import jax.numpy as jnp
import numpy as np
from jax import lax
from jax.experimental import pallas as pl
from jax.experimental.pallas import tpu as pltpu
from jax.experimental.pallas import tpu_sc as plsc

_VOCAB = 1_000_000
_VOCAB_PAD = 1_048_576
_TROWS = 8192
_TGRID = 8
_LANES = 128

_B, _N = 16384, 200
_TOTAL = _B * _N
_NC, _NS = 2, 16
_NW = _NC * _NS
_WCOLS = _B // _NW
_CROWS = 40
_CCOLS = 256
_NCHUNKS = (_N // _CROWS) * (_WCOLS // _CCOLS)
_CHUNK = _CROWS * _CCOLS

_E5 = float(np.exp(5.0))


def _table_body(total_ref, counts_ref, out_ref):
    total = total_ref[0, 0]
    freq = counts_ref[...] / total
    out_ref[...] = 1.0 / (1.0 + (freq + 1e-9) * _E5)


def _transform_table(counts_padded, total_tokens):
    blk = _TROWS // _TGRID
    return pl.pallas_call(
        _table_body,
        grid=(_TGRID,),
        in_specs=[
            pl.BlockSpec(memory_space=pltpu.SMEM),
            pl.BlockSpec((blk, _LANES), lambda i: (i, 0)),
        ],
        out_specs=pl.BlockSpec((blk, _LANES), lambda i: (i, 0)),
        out_shape=jax.ShapeDtypeStruct((_TROWS, _LANES), jnp.float32),
    )(jnp.reshape(total_tokens, (1, 1)), counts_padded)


def _gather_body(xt_hbm, table_hbm, out_hbm, tab_s,
                 idx0, idx1, val0, val1,
                 tsem, isem0, isem1, gsem, osem0, osem1):
    sid = lax.axis_index("s")
    wid = sid * _NC + lax.axis_index("c")
    cbase = wid * _WCOLS

    idx_bufs, val_bufs = [idx0, idx1], [val0, val1]
    isems, osems = [isem0, isem1], [osem0, osem1]

    @pl.when(sid == 0)
    def _():
        pltpu.async_copy(table_hbm, tab_s, tsem).wait()

    def blk(c):
        return (pl.ds((c // 2) * _CROWS, _CROWS),
                pl.ds(cbase + (c % 2) * _CCOLS, _CCOLS))

    idx_loads = [
        pltpu.make_async_copy(xt_hbm.at[blk(c)], idx_bufs[c % 2], isems[c % 2])
        for c in range(_NCHUNKS)
    ]
    out_stores = [
        pltpu.make_async_copy(val_bufs[c % 2], out_hbm.at[blk(c)], osems[c % 2])
        for c in range(_NCHUNKS)
    ]

    idx_loads[0].start()
    plsc.subcore_barrier()

    for c in range(_NCHUNKS):
        b = c % 2
        if c >= 2:
            out_stores[c - 2].wait()
        idx_loads[c].wait()
        if c + 1 < _NCHUNKS:
            idx_loads[c + 1].start()

        def _fire(i, carry):
            r = i >> 1
            k = (i & 1) * 128
            pltpu.make_async_copy(
                tab_s.at[idx_bufs[b].at[r, pl.ds(k, 128)]],
                val_bufs[b].at[r, pl.ds(k, 128)], gsem).start()
            return carry

        lax.fori_loop(0, _CROWS * 2, _fire, 0, unroll=8)

        def _drain(i, carry):
            pltpu.make_async_copy(
                tab_s.at[idx_bufs[b].at[0, pl.ds(0, 128)]],
                val_bufs[b].at[0, pl.ds(0, 128)], gsem).wait()
            return carry

        lax.fori_loop(0, _CROWS * 2, _drain, 0, unroll=8)
        out_stores[c].start()

    out_stores[_NCHUNKS - 2].wait()
    out_stores[_NCHUNKS - 1].wait()


_gather = pl.kernel(
    _gather_body,
    out_type=jax.ShapeDtypeStruct((_N, _B), jnp.float32),
    mesh=plsc.VectorSubcoreMesh(core_axis_name="c", subcore_axis_name="s"),
    scratch_types=[
        pltpu.VMEM_SHARED((_VOCAB_PAD,), jnp.float32),
        pltpu.VMEM((_CROWS, _CCOLS), jnp.int32),
        pltpu.VMEM((_CROWS, _CCOLS), jnp.int32),
        pltpu.VMEM((_CROWS, _CCOLS), jnp.float32),
        pltpu.VMEM((_CROWS, _CCOLS), jnp.float32),
        pltpu.SemaphoreType.DMA,
        pltpu.SemaphoreType.DMA,
        pltpu.SemaphoreType.DMA,
        pltpu.SemaphoreType.DMA,
        pltpu.SemaphoreType.DMA,
        pltpu.SemaphoreType.DMA,
    ],
)


@jax.jit
def kernel(x, token_counts, total_tokens):
    counts_padded = jnp.concatenate(
        [token_counts, jnp.ones((_VOCAB_PAD - _VOCAB,), jnp.float32)]
    ).reshape(_TROWS, _LANES)
    table = _transform_table(counts_padded, total_tokens).reshape(-1)
    out_t = _gather(x.T, table)
    return out_t.T
